# Initial kernel scaffold; baseline (speedup 1.0000x reference)
import jax
import jax.numpy as jnp
from jax.experimental import pallas as pl


def kernel(predict_pc, gt_pc):
    raise NotImplementedError("write your pallas kernel here")


# speedup vs baseline: 1.4682x; 1.4682x over previous
import functools

import jax
import jax.numpy as jnp
from jax.experimental import pallas as pl
from jax.experimental.pallas import tpu as pltpu


def _chamfer_kernel(p_ref, g_ref, out_ref,
                    row_best, row_bestex, col_best, col_bestex, sums,
                    *, nb, ni, nj, denom_m, denom_n):
    b = pl.program_id(0)
    i = pl.program_id(1)
    j = pl.program_id(2)

    p = p_ref[0]
    g = g_ref[0]

    px, py, pz = p[:, 0:1], p[:, 1:2], p[:, 2:3]
    gx, gy, gz = g[0:1, :], g[1:2, :], g[2:3, :]

    dx = px - gx
    dy = py - gy
    dz = pz - gz
    exact = dx * dx + dy * dy + dz * dz

    aa = px * px + py * py + pz * pz
    bb = gx * gx + gy * gy + gz * gz
    ab = jax.lax.dot_general(
        p.astype(jnp.bfloat16), g.astype(jnp.bfloat16),
        (((1,), (0,)), ((), ())), preferred_element_type=jnp.float32)
    approx = (aa + bb) - 2.0 * ab

    inf = jnp.float32(jnp.inf)

    tile_min = jnp.min(approx, axis=1, keepdims=True)
    tile_ex = jnp.min(jnp.where(approx == tile_min, exact, inf),
                      axis=1, keepdims=True)
    prev_min = jnp.where(j == 0, inf, row_best[...])
    prev_ex = jnp.where(j == 0, inf, row_bestex[...])
    upd = tile_min < prev_min
    row_best[...] = jnp.where(upd, tile_min, prev_min)
    row_bestex[...] = jnp.where(upd, tile_ex, prev_ex)

    bn = approx.shape[1]
    csl = (slice(None), pl.ds(j * bn, bn))
    ctile_min = jnp.min(approx, axis=0, keepdims=True)
    ctile_ex = jnp.min(jnp.where(approx == ctile_min, exact, inf),
                       axis=0, keepdims=True)
    cprev_min = jnp.where(i == 0, inf, col_best[csl])
    cprev_ex = jnp.where(i == 0, inf, col_bestex[csl])
    cupd = ctile_min < cprev_min
    col_best[csl] = jnp.where(cupd, ctile_min, cprev_min)
    col_bestex[csl] = jnp.where(cupd, ctile_ex, cprev_ex)

    @pl.when(j == nj - 1)
    def _():
        s = jnp.sum(jnp.sqrt(row_bestex[...] + 1e-8))
        prev = jnp.where((b == 0) & (i == 0), 0.0, sums[0])
        sums[0] = prev + s

    @pl.when(i == ni - 1)
    def _():
        s = jnp.sum(jnp.sqrt(col_bestex[csl] + 1e-8))
        prev = jnp.where((b == 0) & (j == 0), 0.0, sums[1])
        sums[1] = prev + s

    @pl.when((b == nb - 1) & (i == ni - 1) & (j == nj - 1))
    def _():
        loss = sums[0] / denom_m + sums[1] / denom_n
        out_ref[...] = jnp.full((1, 1), loss, jnp.float32)


def _chamfer_call(p_t, gt_pc, bm, bn):
    B, M, _ = p_t.shape
    N = gt_pc.shape[2]
    ni = M // bm
    nj = N // bn
    return pl.pallas_call(
        functools.partial(_chamfer_kernel, nb=B, ni=ni, nj=nj,
                          denom_m=float(B * M), denom_n=float(B * N)),
        grid=(B, ni, nj),
        in_specs=[
            pl.BlockSpec((1, bm, 3), lambda b, i, j: (b, i, 0)),
            pl.BlockSpec((1, 3, bn), lambda b, i, j: (b, 0, j)),
        ],
        out_specs=pl.BlockSpec((1, 1), lambda b, i, j: (0, 0)),
        out_shape=jax.ShapeDtypeStruct((1, 1), jnp.float32),
        scratch_shapes=[
            pltpu.VMEM((bm, 1), jnp.float32),
            pltpu.VMEM((bm, 1), jnp.float32),
            pltpu.VMEM((1, N), jnp.float32),
            pltpu.VMEM((1, N), jnp.float32),
            pltpu.SMEM((2,), jnp.float32),
        ],
    )(p_t, gt_pc)


@jax.jit
def kernel(predict_pc, gt_pc):
    B, _, M = predict_pc.shape
    N = gt_pc.shape[2]
    bm = min(512, M)
    bn = min(1024, N)
    p_t = jnp.swapaxes(predict_pc, 1, 2)
    out = _chamfer_call(p_t, gt_pc, bm, bn)
    return out[0, 0]
